# baseline (speedup 1.0000x reference)
import functools

import jax
import jax.numpy as jnp
import numpy as np
from jax import lax
from jax.experimental import pallas as pl
from jax.experimental.pallas import tpu as pltpu
from jax.experimental.pallas import tpu_sc as plsc

_CH = 80
_NW = 32
_HEADS = 8
_C = 32
_ED = 256


def _head_onehot(ncols, nheads, transpose=False):
    if transpose:
        r = lax.broadcasted_iota(jnp.int32, (nheads, ncols), 1)
        c = lax.broadcasted_iota(jnp.int32, (nheads, ncols), 0)
    else:
        r = lax.broadcasted_iota(jnp.int32, (ncols, nheads), 0)
        c = lax.broadcasted_iota(jnp.int32, (ncols, nheads), 1)
    return (r // _C == c).astype(jnp.float32)


def _dense(a, w, b, relu, bm):
    m, k = a.shape
    n = w.shape[1]

    def kern(a_ref, w_ref, b_ref, o_ref):
        r = jnp.dot(a_ref[...], w_ref[...], preferred_element_type=jnp.float32)
        r = r + b_ref[...]
        o_ref[...] = jnp.maximum(r, 0.0) if relu else r

    return pl.pallas_call(
        kern,
        grid=(m // bm,),
        in_specs=[pl.BlockSpec((bm, k), lambda i: (i, 0)),
                  pl.BlockSpec((k, n), lambda i: (0, 0)),
                  pl.BlockSpec((1, n), lambda i: (0, 0))],
        out_specs=pl.BlockSpec((bm, n), lambda i: (i, 0)),
        out_shape=jax.ShapeDtypeStruct((m, n), jnp.float32),
    )(a, w, b.reshape(1, n))


def _proj(h, w4, b4, bm):
    m = h.shape[0]

    def kern(h_ref, w_ref, b_ref, q_ref, k_ref, v0_ref, v1_ref, s_ref):
        p = jnp.dot(h_ref[...], w_ref[...], preferred_element_type=jnp.float32)
        p = p + b_ref[...]
        q_ref[...] = p[:, 0:256]
        k_ref[...] = p[:, 256:512]
        v0_ref[...] = p[:, 512:640]
        v1_ref[...] = p[:, 640:768]
        s_ref[...] = p[:, 768:1024]

    shp = jax.ShapeDtypeStruct((m, _ED), jnp.float32)
    hshp = jax.ShapeDtypeStruct((m, 128), jnp.float32)
    return pl.pallas_call(
        kern,
        grid=(m // bm,),
        in_specs=[pl.BlockSpec((bm, _ED), lambda i: (i, 0)),
                  pl.BlockSpec((_ED, 4 * _ED), lambda i: (0, 0)),
                  pl.BlockSpec((1, 4 * _ED), lambda i: (0, 0))],
        out_specs=[pl.BlockSpec((bm, _ED), lambda i: (i, 0)),
                   pl.BlockSpec((bm, _ED), lambda i: (i, 0)),
                   pl.BlockSpec((bm, 128), lambda i: (i, 0)),
                   pl.BlockSpec((bm, 128), lambda i: (i, 0)),
                   pl.BlockSpec((bm, _ED), lambda i: (i, 0))],
        out_shape=[shp, shp, hshp, hshp, shp],
    )(h, w4, b4.reshape(1, 4 * _ED))


def _kmax(k):
    m = k.shape[0]

    def kern(k_ref, o_ref):
        kk = k_ref[...]
        g = _head_onehot(_ED, _HEADS)
        kn2 = jnp.dot(kk * kk, g, preferred_element_type=jnp.float32)
        o_ref[...] = jnp.sqrt(jnp.max(kn2, axis=0, keepdims=True) / float(_C))

    return pl.pallas_call(
        kern,
        out_shape=jax.ShapeDtypeStruct((1, _HEADS), jnp.float32),
    )(k)


def _edge_math(qd, ks, kmaxs, be):
    etot = qd.shape[0]
    inv = 1.0 / float(np.sqrt(_C))

    def kern(qd_ref, ks_ref, km_ref, ex_ref):
        g = _head_onehot(_ED, _HEADS)
        q = qd_ref[...]
        k = ks_ref[...]
        score = jnp.dot(q * k, g, preferred_element_type=jnp.float32) * inv
        qn2 = jnp.dot(q * q, g, preferred_element_type=jnp.float32)
        mbound = jnp.sqrt(qn2) * km_ref[...]
        ex = jnp.exp(score - mbound)
        r16 = lax.broadcasted_iota(jnp.int32, (_HEADS, 128), 0)
        c16 = lax.broadcasted_iota(jnp.int32, (_HEADS, 128), 1)
        g16 = (c16 // 16 == r16).astype(jnp.float32)
        ex_ref[...] = jnp.dot(ex, g16, preferred_element_type=jnp.float32)

    return pl.pallas_call(
        kern,
        grid=(etot // be,),
        in_specs=[pl.BlockSpec((be, _ED), lambda i: (i, 0)),
                  pl.BlockSpec((be, _ED), lambda i: (i, 0)),
                  pl.BlockSpec((1, _HEADS), lambda i: (0, 0))],
        out_specs=pl.BlockSpec((be, 128), lambda i: (i, 0)),
        out_shape=jax.ShapeDtypeStruct((etot, 128), jnp.float32),
    )(qd, ks, kmaxs)


def _combine(out0, out1, den0, den1, s, bm):
    m = s.shape[0]

    def kern(o0_ref, o1_ref, d0_ref, d1_ref, s_ref, h_ref):
        r = lax.broadcasted_iota(jnp.int32, (128, 128), 0)
        c = lax.broadcasted_iota(jnp.int32, (128, 128), 1)
        realign = jnp.where(r // 16 == c // 16, 1.0 / 16.0, 0.0)
        d = d0_ref[...] + d1_ref[...]
        dexp = jnp.dot(d, realign, preferred_element_type=jnp.float32) + 1e-16
        attn = jnp.concatenate([o0_ref[...] / dexp, o1_ref[...] / dexp], axis=1)
        rp = lax.broadcasted_iota(jnp.int32, (_ED, _ED), 0)
        cp = lax.broadcasted_iota(jnp.int32, (_ED, _ED), 1)
        orig = 32 * ((rp % 128) // 16) + (rp % 16) + 16 * (rp // 128)
        pmat = (cp == orig).astype(jnp.float32)
        h_ref[...] = jnp.dot(attn, pmat,
                             preferred_element_type=jnp.float32) + s_ref[...]

    return pl.pallas_call(
        kern,
        grid=(m // bm,),
        in_specs=[pl.BlockSpec((bm, 128), lambda i: (i, 0)),
                  pl.BlockSpec((bm, 128), lambda i: (i, 0)),
                  pl.BlockSpec((bm, 128), lambda i: (i, 0)),
                  pl.BlockSpec((bm, 128), lambda i: (i, 0)),
                  pl.BlockSpec((bm, _ED), lambda i: (i, 0))],
        out_specs=pl.BlockSpec((bm, _ED), lambda i: (i, 0)),
        out_shape=jax.ShapeDtypeStruct((m, _ED), jnp.float32),
    )(out0, out1, den0, den1, s)


def _final(h, w, b, bm):
    m = h.shape[0]
    n = w.shape[1]

    def kern(h_ref, w_ref, b_ref, o_ref):
        o = jnp.dot(h_ref[...], w_ref[...], preferred_element_type=jnp.float32)
        o = o + b_ref[...]
        norm = jnp.sqrt(jnp.sum(o * o, axis=1, keepdims=True))
        o_ref[...] = o / jnp.maximum(norm, 1e-12)

    return pl.pallas_call(
        kern,
        grid=(m // bm,),
        in_specs=[pl.BlockSpec((bm, _ED), lambda i: (i, 0)),
                  pl.BlockSpec((_ED, n), lambda i: (0, 0)),
                  pl.BlockSpec((1, n), lambda i: (0, 0))],
        out_specs=pl.BlockSpec((bm, n), lambda i: (i, 0)),
        out_shape=jax.ShapeDtypeStruct((m, n), jnp.float32),
    )(h, w, b.reshape(1, n))


def _gather2(q, k, dst, src):
    etot = dst.shape[0]
    per_w = etot // _NW
    nch = per_w // _CH
    mesh = plsc.VectorSubcoreMesh(core_axis_name="c", subcore_axis_name="s")
    oshp = jax.ShapeDtypeStruct((etot, _ED), jnp.float32)

    @functools.partial(
        pl.kernel, mesh=mesh,
        out_type=[oshp, oshp],
        scratch_types=[pltpu.VMEM((_CH,), jnp.int32),
                       pltpu.VMEM((_CH,), jnp.int32),
                       pltpu.VMEM((_CH, _ED), jnp.float32),
                       pltpu.VMEM((_CH, _ED), jnp.float32),
                       pltpu.SemaphoreType.DMA,
                       pltpu.SemaphoreType.DMA])
    def kern(q_hbm, k_hbm, dst_hbm, src_hbm, qd_hbm, ks_hbm,
             di_v, si_v, qbuf, kbuf, sem1, sem2):
        wid = lax.axis_index("s") * 2 + lax.axis_index("c")
        base0 = wid * per_w

        def body(j, carry):
            base = base0 + j * _CH
            pltpu.sync_copy(dst_hbm.at[pl.ds(base, _CH)], di_v)
            pltpu.sync_copy(src_hbm.at[pl.ds(base, _CH)], si_v)
            c1 = pltpu.async_copy(q_hbm.at[di_v], qbuf, sem1)
            c2 = pltpu.async_copy(k_hbm.at[si_v], kbuf, sem2)
            c1.wait()
            c2.wait()
            pltpu.sync_copy(qbuf, qd_hbm.at[pl.ds(base, _CH)])
            pltpu.sync_copy(kbuf, ks_hbm.at[pl.ds(base, _CH)])
            return carry

        lax.fori_loop(0, nch, body, 0)

    return kern(q, k, dst, src)


def _scatter(v0, v1, ex, dst, src, np_):
    etot = dst.shape[0]
    per_s = etot // 16
    nch = per_s // _CH
    per_s2 = etot // 32
    nch2 = per_s2 // _CH
    rows_per_sub = np_ // 16
    mesh = plsc.VectorSubcoreMesh(core_axis_name="c", subcore_axis_name="s")
    zrows = jnp.zeros((16, 128), jnp.float32)
    oshp = jax.ShapeDtypeStruct((np_, 128), jnp.float32)

    @functools.partial(
        pl.kernel, mesh=mesh,
        out_type=[oshp, oshp, oshp, oshp],
        scratch_types=[pltpu.VMEM((_CH,), jnp.int32),
                       pltpu.VMEM((_CH,), jnp.int32),
                       pltpu.VMEM((_CH, 128), jnp.float32),
                       pltpu.VMEM((_CH, 128), jnp.float32),
                       pltpu.VMEM((16, 128), jnp.float32),
                       pltpu.VMEM_SHARED((np_, 128), jnp.float32),
                       pltpu.SemaphoreType.DMA])
    def kern(v0_hbm, v1_hbm, ex_hbm, dst_hbm, src_hbm, z_hbm,
             out0_hbm, out1_hbm, den0_hbm, den1_hbm,
             di_v, si_v, vbuf, exbuf, zbuf, acc_sh, sem):
        cc = lax.axis_index("c")
        ss = lax.axis_index("s")
        row0 = ss * rows_per_sub

        pltpu.sync_copy(z_hbm, zbuf)

        def zero_acc():
            def zbody(t, carry):
                pltpu.sync_copy(zbuf, acc_sh.at[pl.ds(row0 + t * 16, 16)])
                return carry
            lax.fori_loop(0, rows_per_sub // 16, zbody, 0)

        def accum_v(v_hbm):
            def body(j, carry):
                base = ss * per_s + j * _CH
                pltpu.sync_copy(dst_hbm.at[pl.ds(base, _CH)], di_v)
                pltpu.sync_copy(src_hbm.at[pl.ds(base, _CH)], si_v)
                cpy = pltpu.async_copy(v_hbm.at[si_v], vbuf, sem)
                pltpu.sync_copy(ex_hbm.at[pl.ds(base, _CH)], exbuf)
                cpy.wait()

                def mul_row(e, carry2):
                    for g in range(8):
                        sl = pl.ds(16 * g, 16)
                        vbuf[e, sl] = vbuf[e, sl] * exbuf[e, sl]
                    return carry2

                lax.fori_loop(0, _CH, mul_row, 0)
                pltpu.sync_copy(vbuf, acc_sh.at[di_v], add=True)
                return carry

            lax.fori_loop(0, nch, body, 0)

        def accum_ex(base0):
            def body(j, carry):
                base = base0 + j * _CH
                pltpu.sync_copy(dst_hbm.at[pl.ds(base, _CH)], di_v)
                pltpu.sync_copy(ex_hbm.at[pl.ds(base, _CH)], exbuf)
                pltpu.sync_copy(exbuf, acc_sh.at[di_v], add=True)
                return carry
            lax.fori_loop(0, nch2, body, 0)

        def dump(dst_hbm_out):
            pltpu.sync_copy(acc_sh.at[pl.ds(row0, rows_per_sub)],
                            dst_hbm_out.at[pl.ds(row0, rows_per_sub)])

        zero_acc()
        plsc.subcore_barrier()

        @pl.when(cc == 0)
        def _():
            accum_v(v0_hbm)

        @pl.when(cc == 1)
        def _():
            accum_v(v1_hbm)

        plsc.subcore_barrier()

        @pl.when(cc == 0)
        def _():
            dump(out0_hbm)

        @pl.when(cc == 1)
        def _():
            dump(out1_hbm)

        plsc.subcore_barrier()

        zero_acc()
        plsc.subcore_barrier()

        @pl.when(cc == 0)
        def _():
            accum_ex(ss * per_s2)

        @pl.when(cc == 1)
        def _():
            accum_ex(etot // 2 + ss * per_s2)

        plsc.subcore_barrier()

        @pl.when(cc == 0)
        def _():
            dump(den0_hbm)

        @pl.when(cc == 1)
        def _():
            dump(den1_hbm)

    return kern(v0, v1, ex, dst, src, zrows)


def kernel(x, edge_index, w_in1, b_in1, w_in2, b_in2, w_in3, b_in3,
           wq, bq, wk, bk, wv, bv, ws, bs,
           w_o1, b_o1, w_o2, b_o2, w_o3, b_o3):
    n = x.shape[0]
    npad = ((n + 511) // 512) * 512
    nlayers = wq.shape[0]
    bm = 512

    xp = jnp.pad(x, ((0, npad - n), (0, 0)))
    src = edge_index[0]
    dst = edge_index[1]

    h = _dense(xp, w_in1, b_in1, True, bm)
    h = _dense(h, w_in2, b_in2, True, bm)
    h = _dense(h, w_in3, b_in3, True, bm)

    pperm = np.empty((_ED,), np.int32)
    for p in range(_ED):
        pperm[p] = 32 * ((p % 128) // 16) + (p % 16) + 16 * (p // 128)

    for l in range(nlayers):
        w4 = jnp.concatenate([wq[l], wk[l], wv[l][:, pperm], ws[l]], axis=1)
        b4 = jnp.concatenate([bq[l], bk[l], bv[l][pperm], bs[l]])
        q, k, v0, v1, s = _proj(h, w4, b4, bm)
        kmaxs = _kmax(k)
        qd, ksg = _gather2(q, k, dst, src)
        ex = _edge_math(qd, ksg, kmaxs, 1000)
        out0, out1, den0, den1 = _scatter(v0, v1, ex, dst, src, npad)
        h = _combine(out0, out1, den0, den1, s, bm)

    h = _dense(h, w_o1, b_o1, True, bm)
    h = _dense(h, w_o2, b_o2, True, bm)
    o = _final(h, w_o3, b_o3, bm)
    return o[:n]


# speedup vs baseline: 1.2766x; 1.2766x over previous
import functools

import jax
import jax.numpy as jnp
import numpy as np
from jax import lax
from jax.experimental import pallas as pl
from jax.experimental.pallas import tpu as pltpu
from jax.experimental.pallas import tpu_sc as plsc

_CH = 80
_NW = 32
_HEADS = 8
_C = 32
_ED = 256


def _head_onehot(ncols, nheads, transpose=False):
    if transpose:
        r = lax.broadcasted_iota(jnp.int32, (nheads, ncols), 1)
        c = lax.broadcasted_iota(jnp.int32, (nheads, ncols), 0)
    else:
        r = lax.broadcasted_iota(jnp.int32, (ncols, nheads), 0)
        c = lax.broadcasted_iota(jnp.int32, (ncols, nheads), 1)
    return (r // _C == c).astype(jnp.float32)


def _dense(a, w, b, relu, bm):
    m, k = a.shape
    n = w.shape[1]

    def kern(a_ref, w_ref, b_ref, o_ref):
        r = jnp.dot(a_ref[...], w_ref[...], preferred_element_type=jnp.float32)
        r = r + b_ref[...]
        o_ref[...] = jnp.maximum(r, 0.0) if relu else r

    return pl.pallas_call(
        kern,
        grid=(m // bm,),
        in_specs=[pl.BlockSpec((bm, k), lambda i: (i, 0)),
                  pl.BlockSpec((k, n), lambda i: (0, 0)),
                  pl.BlockSpec((1, n), lambda i: (0, 0))],
        out_specs=pl.BlockSpec((bm, n), lambda i: (i, 0)),
        out_shape=jax.ShapeDtypeStruct((m, n), jnp.float32),
    )(a, w, b.reshape(1, n))


def _proj(h, w4, b4, bm):
    m = h.shape[0]

    def kern(h_ref, w_ref, b_ref, q_ref, k_ref, v0_ref, v1_ref, s_ref):
        p = jnp.dot(h_ref[...], w_ref[...], preferred_element_type=jnp.float32)
        p = p + b_ref[...]
        q_ref[...] = p[:, 0:256]
        k_ref[...] = p[:, 256:512]
        v0_ref[...] = p[:, 512:640]
        v1_ref[...] = p[:, 640:768]
        s_ref[...] = p[:, 768:1024]

    shp = jax.ShapeDtypeStruct((m, _ED), jnp.float32)
    hshp = jax.ShapeDtypeStruct((m, 128), jnp.float32)
    return pl.pallas_call(
        kern,
        grid=(m // bm,),
        in_specs=[pl.BlockSpec((bm, _ED), lambda i: (i, 0)),
                  pl.BlockSpec((_ED, 4 * _ED), lambda i: (0, 0)),
                  pl.BlockSpec((1, 4 * _ED), lambda i: (0, 0))],
        out_specs=[pl.BlockSpec((bm, _ED), lambda i: (i, 0)),
                   pl.BlockSpec((bm, _ED), lambda i: (i, 0)),
                   pl.BlockSpec((bm, 128), lambda i: (i, 0)),
                   pl.BlockSpec((bm, 128), lambda i: (i, 0)),
                   pl.BlockSpec((bm, _ED), lambda i: (i, 0))],
        out_shape=[shp, shp, hshp, hshp, shp],
    )(h, w4, b4.reshape(1, 4 * _ED))


def _kmax(k):
    m = k.shape[0]

    def kern(k_ref, o_ref):
        kk = k_ref[...]
        g = _head_onehot(_ED, _HEADS)
        kn2 = jnp.dot(kk * kk, g, preferred_element_type=jnp.float32)
        o_ref[...] = jnp.sqrt(jnp.max(kn2, axis=0, keepdims=True) / float(_C))

    return pl.pallas_call(
        kern,
        out_shape=jax.ShapeDtypeStruct((1, _HEADS), jnp.float32),
    )(k)


def _edge_math(qd, ks, kmaxs, be):
    etot = qd.shape[0]
    inv = 1.0 / float(np.sqrt(_C))

    def kern(qd_ref, ks_ref, km_ref, ex_ref):
        g = _head_onehot(_ED, _HEADS)
        q = qd_ref[...]
        k = ks_ref[...]
        score = jnp.dot(q * k, g, preferred_element_type=jnp.float32) * inv
        qn2 = jnp.dot(q * q, g, preferred_element_type=jnp.float32)
        mbound = jnp.sqrt(qn2) * km_ref[...]
        ex = jnp.exp(score - mbound)
        r16 = lax.broadcasted_iota(jnp.int32, (_HEADS, 128), 0)
        c16 = lax.broadcasted_iota(jnp.int32, (_HEADS, 128), 1)
        g16 = (c16 // 16 == r16).astype(jnp.float32)
        ex_ref[...] = jnp.dot(ex, g16, preferred_element_type=jnp.float32)

    return pl.pallas_call(
        kern,
        grid=(etot // be,),
        in_specs=[pl.BlockSpec((be, _ED), lambda i: (i, 0)),
                  pl.BlockSpec((be, _ED), lambda i: (i, 0)),
                  pl.BlockSpec((1, _HEADS), lambda i: (0, 0))],
        out_specs=pl.BlockSpec((be, 128), lambda i: (i, 0)),
        out_shape=jax.ShapeDtypeStruct((etot, 128), jnp.float32),
    )(qd, ks, kmaxs)


def _combine(out0, out1, den0, den1, s, bm):
    m = s.shape[0]

    def kern(o0_ref, o1_ref, d0_ref, d1_ref, s_ref, h_ref):
        r = lax.broadcasted_iota(jnp.int32, (128, 128), 0)
        c = lax.broadcasted_iota(jnp.int32, (128, 128), 1)
        realign = jnp.where(r // 16 == c // 16, 1.0 / 16.0, 0.0)
        d = d0_ref[...] + d1_ref[...]
        dexp = jnp.dot(d, realign, preferred_element_type=jnp.float32) + 1e-16
        attn = jnp.concatenate([o0_ref[...] / dexp, o1_ref[...] / dexp], axis=1)
        rp = lax.broadcasted_iota(jnp.int32, (_ED, _ED), 0)
        cp = lax.broadcasted_iota(jnp.int32, (_ED, _ED), 1)
        orig = 32 * ((rp % 128) // 16) + (rp % 16) + 16 * (rp // 128)
        pmat = (cp == orig).astype(jnp.float32)
        h_ref[...] = jnp.dot(attn, pmat,
                             preferred_element_type=jnp.float32) + s_ref[...]

    return pl.pallas_call(
        kern,
        grid=(m // bm,),
        in_specs=[pl.BlockSpec((bm, 128), lambda i: (i, 0)),
                  pl.BlockSpec((bm, 128), lambda i: (i, 0)),
                  pl.BlockSpec((bm, 128), lambda i: (i, 0)),
                  pl.BlockSpec((bm, 128), lambda i: (i, 0)),
                  pl.BlockSpec((bm, _ED), lambda i: (i, 0))],
        out_specs=pl.BlockSpec((bm, _ED), lambda i: (i, 0)),
        out_shape=jax.ShapeDtypeStruct((m, _ED), jnp.float32),
    )(out0, out1, den0, den1, s)


def _final(h, w, b, bm):
    m = h.shape[0]
    n = w.shape[1]

    def kern(h_ref, w_ref, b_ref, o_ref):
        o = jnp.dot(h_ref[...], w_ref[...], preferred_element_type=jnp.float32)
        o = o + b_ref[...]
        norm = jnp.sqrt(jnp.sum(o * o, axis=1, keepdims=True))
        o_ref[...] = o / jnp.maximum(norm, 1e-12)

    return pl.pallas_call(
        kern,
        grid=(m // bm,),
        in_specs=[pl.BlockSpec((bm, _ED), lambda i: (i, 0)),
                  pl.BlockSpec((_ED, n), lambda i: (0, 0)),
                  pl.BlockSpec((1, n), lambda i: (0, 0))],
        out_specs=pl.BlockSpec((bm, n), lambda i: (i, 0)),
        out_shape=jax.ShapeDtypeStruct((m, n), jnp.float32),
    )(h, w, b.reshape(1, n))


def _gather2(q, k, dst, src):
    etot = dst.shape[0]
    per_w = etot // _NW
    nch = per_w // _CH
    mesh = plsc.VectorSubcoreMesh(core_axis_name="c", subcore_axis_name="s")
    oshp = jax.ShapeDtypeStruct((etot, _ED), jnp.float32)

    @functools.partial(
        pl.kernel, mesh=mesh,
        out_type=[oshp, oshp],
        scratch_types=[pltpu.VMEM((_CH,), jnp.int32),
                       pltpu.VMEM((_CH,), jnp.int32),
                       pltpu.VMEM((_CH,), jnp.int32),
                       pltpu.VMEM((_CH,), jnp.int32),
                       pltpu.VMEM((_CH, _ED), jnp.float32),
                       pltpu.VMEM((_CH, _ED), jnp.float32),
                       pltpu.VMEM((_CH, _ED), jnp.float32),
                       pltpu.VMEM((_CH, _ED), jnp.float32),
                       pltpu.SemaphoreType.DMA,
                       pltpu.SemaphoreType.DMA])
    def kern(q_hbm, k_hbm, dst_hbm, src_hbm, qd_hbm, ks_hbm,
             di_a, si_a, di_b, si_b, qbuf_a, kbuf_a, qbuf_b, kbuf_b,
             sem_a, sem_b):
        wid = lax.axis_index("s") * 2 + lax.axis_index("c")
        base0 = wid * per_w

        def fire(base, di_v, si_v, qbuf, kbuf, sem):
            pltpu.sync_copy(dst_hbm.at[pl.ds(base, _CH)], di_v)
            pltpu.sync_copy(src_hbm.at[pl.ds(base, _CH)], si_v)
            c1 = pltpu.async_copy(q_hbm.at[di_v], qbuf, sem)
            c2 = pltpu.async_copy(k_hbm.at[si_v], kbuf, sem)
            return c1, c2

        def drain(base, cpys, qbuf, kbuf):
            c1, c2 = cpys
            c1.wait()
            c2.wait()
            pltpu.sync_copy(qbuf, qd_hbm.at[pl.ds(base, _CH)])
            pltpu.sync_copy(kbuf, ks_hbm.at[pl.ds(base, _CH)])

        def pair(t, carry):
            base_a = base0 + (2 * t) * _CH
            base_b = base_a + _CH
            ca = fire(base_a, di_a, si_a, qbuf_a, kbuf_a, sem_a)
            cb = fire(base_b, di_b, si_b, qbuf_b, kbuf_b, sem_b)
            drain(base_a, ca, qbuf_a, kbuf_a)
            drain(base_b, cb, qbuf_b, kbuf_b)
            return carry

        lax.fori_loop(0, nch // 2, pair, 0)
        if nch % 2:
            base_t = base0 + (nch - 1) * _CH
            ct = fire(base_t, di_a, si_a, qbuf_a, kbuf_a, sem_a)
            drain(base_t, ct, qbuf_a, kbuf_a)

    return kern(q, k, dst, src)


def _scatter(v0, v1, ex, dst, src, np_):
    etot = dst.shape[0]
    per_s = etot // 16
    nch = per_s // _CH
    per_s2 = etot // 32
    nch2 = per_s2 // _CH
    rows_per_sub = np_ // 16
    mesh = plsc.VectorSubcoreMesh(core_axis_name="c", subcore_axis_name="s")
    zrows = jnp.zeros((16, 128), jnp.float32)
    oshp = jax.ShapeDtypeStruct((np_, 128), jnp.float32)

    @functools.partial(
        pl.kernel, mesh=mesh,
        out_type=[oshp, oshp, oshp, oshp],
        scratch_types=[pltpu.VMEM((_CH,), jnp.int32),
                       pltpu.VMEM((_CH,), jnp.int32),
                       pltpu.VMEM((_CH,), jnp.int32),
                       pltpu.VMEM((_CH,), jnp.int32),
                       pltpu.VMEM((_CH, 128), jnp.float32),
                       pltpu.VMEM((_CH, 128), jnp.float32),
                       pltpu.VMEM((_CH, 128), jnp.float32),
                       pltpu.VMEM((_CH, 128), jnp.float32),
                       pltpu.VMEM((16, 128), jnp.float32),
                       pltpu.VMEM_SHARED((np_, 128), jnp.float32),
                       pltpu.SemaphoreType.DMA,
                       pltpu.SemaphoreType.DMA])
    def kern(v0_hbm, v1_hbm, ex_hbm, dst_hbm, src_hbm, z_hbm,
             out0_hbm, out1_hbm, den0_hbm, den1_hbm,
             di_a, si_a, di_b, si_b, vbuf_a, exbuf_a, vbuf_b, exbuf_b,
             zbuf, acc_sh, sem_a, sem_b):
        cc = lax.axis_index("c")
        ss = lax.axis_index("s")
        row0 = ss * rows_per_sub

        pltpu.sync_copy(z_hbm, zbuf)

        def zero_acc():
            def zbody(t, carry):
                pltpu.sync_copy(zbuf, acc_sh.at[pl.ds(row0 + t * 16, 16)])
                return carry
            lax.fori_loop(0, rows_per_sub // 16, zbody, 0)

        def mul_rows(vbuf, exbuf):
            def mul_row(e, carry2):
                for g in range(8):
                    sl = pl.ds(16 * g, 16)
                    vbuf[e, sl] = vbuf[e, sl] * exbuf[e, sl]
                return carry2
            lax.fori_loop(0, _CH, mul_row, 0)

        def accum_v(v_hbm):
            def fire(base, di_v, si_v, vbuf, exbuf, sem):
                pltpu.sync_copy(dst_hbm.at[pl.ds(base, _CH)], di_v)
                pltpu.sync_copy(src_hbm.at[pl.ds(base, _CH)], si_v)
                c1 = pltpu.async_copy(v_hbm.at[si_v], vbuf, sem)
                c2 = pltpu.async_copy(ex_hbm.at[pl.ds(base, _CH)], exbuf, sem)
                return c1, c2

            def proc(cpys, di_v, vbuf, exbuf):
                c1, c2 = cpys
                c1.wait()
                c2.wait()
                mul_rows(vbuf, exbuf)
                pltpu.sync_copy(vbuf, acc_sh.at[di_v], add=True)

            def pair(t, carry):
                base_a = ss * per_s + (2 * t) * _CH
                ca = fire(base_a, di_a, si_a, vbuf_a, exbuf_a, sem_a)
                cb = fire(base_a + _CH, di_b, si_b, vbuf_b, exbuf_b, sem_b)
                proc(ca, di_a, vbuf_a, exbuf_a)
                proc(cb, di_b, vbuf_b, exbuf_b)
                return carry

            lax.fori_loop(0, nch // 2, pair, 0)
            if nch % 2:
                base_t = ss * per_s + (nch - 1) * _CH
                ct = fire(base_t, di_a, si_a, vbuf_a, exbuf_a, sem_a)
                proc(ct, di_a, vbuf_a, exbuf_a)

        def accum_ex(base0):
            def fire(base, di_v, exbuf, sem):
                pltpu.sync_copy(dst_hbm.at[pl.ds(base, _CH)], di_v)
                return pltpu.async_copy(ex_hbm.at[pl.ds(base, _CH)], exbuf, sem)

            def pair(t, carry):
                base_a = base0 + (2 * t) * _CH
                ca = fire(base_a, di_a, exbuf_a, sem_a)
                cb = fire(base_a + _CH, di_b, exbuf_b, sem_b)
                ca.wait()
                pltpu.sync_copy(exbuf_a, acc_sh.at[di_a], add=True)
                cb.wait()
                pltpu.sync_copy(exbuf_b, acc_sh.at[di_b], add=True)
                return carry

            lax.fori_loop(0, nch2 // 2, pair, 0)
            if nch2 % 2:
                base_t = base0 + (nch2 - 1) * _CH
                ct = fire(base_t, di_a, exbuf_a, sem_a)
                ct.wait()
                pltpu.sync_copy(exbuf_a, acc_sh.at[di_a], add=True)

        def dump(dst_hbm_out):
            pltpu.sync_copy(acc_sh.at[pl.ds(row0, rows_per_sub)],
                            dst_hbm_out.at[pl.ds(row0, rows_per_sub)])

        zero_acc()
        plsc.subcore_barrier()

        @pl.when(cc == 0)
        def _():
            accum_v(v0_hbm)

        @pl.when(cc == 1)
        def _():
            accum_v(v1_hbm)

        plsc.subcore_barrier()

        @pl.when(cc == 0)
        def _():
            dump(out0_hbm)

        @pl.when(cc == 1)
        def _():
            dump(out1_hbm)

        plsc.subcore_barrier()

        zero_acc()
        plsc.subcore_barrier()

        @pl.when(cc == 0)
        def _():
            accum_ex(ss * per_s2)

        @pl.when(cc == 1)
        def _():
            accum_ex(etot // 2 + ss * per_s2)

        plsc.subcore_barrier()

        @pl.when(cc == 0)
        def _():
            dump(den0_hbm)

        @pl.when(cc == 1)
        def _():
            dump(den1_hbm)

    return kern(v0, v1, ex, dst, src, zrows)


def kernel(x, edge_index, w_in1, b_in1, w_in2, b_in2, w_in3, b_in3,
           wq, bq, wk, bk, wv, bv, ws, bs,
           w_o1, b_o1, w_o2, b_o2, w_o3, b_o3):
    n = x.shape[0]
    npad = ((n + 511) // 512) * 512
    nlayers = wq.shape[0]
    bm = 512

    xp = jnp.pad(x, ((0, npad - n), (0, 0)))
    src = edge_index[0]
    dst = edge_index[1]

    h = _dense(xp, w_in1, b_in1, True, bm)
    h = _dense(h, w_in2, b_in2, True, bm)
    h = _dense(h, w_in3, b_in3, True, bm)

    pperm = np.empty((_ED,), np.int32)
    for p in range(_ED):
        pperm[p] = 32 * ((p % 128) // 16) + (p % 16) + 16 * (p // 128)

    for l in range(nlayers):
        w4 = jnp.concatenate([wq[l], wk[l], wv[l][:, pperm], ws[l]], axis=1)
        b4 = jnp.concatenate([bq[l], bk[l], bv[l][pperm], bs[l]])
        q, k, v0, v1, s = _proj(h, w4, b4, bm)
        kmaxs = _kmax(k)
        qd, ksg = _gather2(q, k, dst, src)
        ex = _edge_math(qd, ksg, kmaxs, 1000)
        out0, out1, den0, den1 = _scatter(v0, v1, ex, dst, src, npad)
        h = _combine(out0, out1, den0, den1, s, bm)

    h = _dense(h, w_o1, b_o1, True, bm)
    h = _dense(h, w_o2, b_o2, True, bm)
    o = _final(h, w_o3, b_o3, bm)
    return o[:n]


# speedup vs baseline: 1.3111x; 1.0270x over previous
import functools

import jax
import jax.numpy as jnp
import numpy as np
from jax import lax
from jax.experimental import pallas as pl
from jax.experimental.pallas import tpu as pltpu
from jax.experimental.pallas import tpu_sc as plsc

_CH = 80
_NW = 32
_HEADS = 8
_C = 32
_ED = 256


def _head_onehot(ncols, nheads, transpose=False):
    if transpose:
        r = lax.broadcasted_iota(jnp.int32, (nheads, ncols), 1)
        c = lax.broadcasted_iota(jnp.int32, (nheads, ncols), 0)
    else:
        r = lax.broadcasted_iota(jnp.int32, (ncols, nheads), 0)
        c = lax.broadcasted_iota(jnp.int32, (ncols, nheads), 1)
    return (r // _C == c).astype(jnp.float32)


def _mlp3(a, w1, b1, w2, b2, w3, b3, bm, final_norm=False):
    m, k = a.shape
    n = w3.shape[1]

    def kern(a_ref, w1_ref, b1_ref, w2_ref, b2_ref, w3_ref, b3_ref, o_ref):
        h1 = jnp.maximum(jnp.dot(a_ref[...], w1_ref[...],
                                 preferred_element_type=jnp.float32)
                         + b1_ref[...], 0.0)
        h2 = jnp.maximum(jnp.dot(h1, w2_ref[...],
                                 preferred_element_type=jnp.float32)
                         + b2_ref[...], 0.0)
        o = jnp.dot(h2, w3_ref[...],
                    preferred_element_type=jnp.float32) + b3_ref[...]
        if final_norm:
            norm = jnp.sqrt(jnp.sum(o * o, axis=1, keepdims=True))
            o_ref[...] = o / jnp.maximum(norm, 1e-12)
        else:
            o_ref[...] = jnp.maximum(o, 0.0)

    k2 = w2.shape[0]
    k3 = w3.shape[0]
    return pl.pallas_call(
        kern,
        grid=(m // bm,),
        in_specs=[pl.BlockSpec((bm, k), lambda i: (i, 0)),
                  pl.BlockSpec((k, k2), lambda i: (0, 0)),
                  pl.BlockSpec((1, k2), lambda i: (0, 0)),
                  pl.BlockSpec((k2, k3), lambda i: (0, 0)),
                  pl.BlockSpec((1, k3), lambda i: (0, 0)),
                  pl.BlockSpec((k3, n), lambda i: (0, 0)),
                  pl.BlockSpec((1, n), lambda i: (0, 0))],
        out_specs=pl.BlockSpec((bm, n), lambda i: (i, 0)),
        out_shape=jax.ShapeDtypeStruct((m, n), jnp.float32),
    )(a, w1, b1.reshape(1, -1), w2, b2.reshape(1, -1), w3, b3.reshape(1, -1))


def _proj(h, w4, b4, bm):
    m = h.shape[0]

    def kern(h_ref, w_ref, b_ref, q_ref, k_ref, v0_ref, v1_ref, s_ref):
        p = jnp.dot(h_ref[...], w_ref[...], preferred_element_type=jnp.float32)
        p = p + b_ref[...]
        q_ref[...] = p[:, 0:256]
        k_ref[...] = p[:, 256:512]
        v0_ref[...] = p[:, 512:640]
        v1_ref[...] = p[:, 640:768]
        s_ref[...] = p[:, 768:1024]

    shp = jax.ShapeDtypeStruct((m, _ED), jnp.float32)
    hshp = jax.ShapeDtypeStruct((m, 128), jnp.float32)
    return pl.pallas_call(
        kern,
        grid=(m // bm,),
        in_specs=[pl.BlockSpec((bm, _ED), lambda i: (i, 0)),
                  pl.BlockSpec((_ED, 4 * _ED), lambda i: (0, 0)),
                  pl.BlockSpec((1, 4 * _ED), lambda i: (0, 0))],
        out_specs=[pl.BlockSpec((bm, _ED), lambda i: (i, 0)),
                   pl.BlockSpec((bm, _ED), lambda i: (i, 0)),
                   pl.BlockSpec((bm, 128), lambda i: (i, 0)),
                   pl.BlockSpec((bm, 128), lambda i: (i, 0)),
                   pl.BlockSpec((bm, _ED), lambda i: (i, 0))],
        out_shape=[shp, shp, hshp, hshp, shp],
    )(h, w4, b4.reshape(1, 4 * _ED))


def _kmax(k):
    m = k.shape[0]

    def kern(k_ref, o_ref):
        kk = k_ref[...]
        g = _head_onehot(_ED, _HEADS)
        kn2 = jnp.dot(kk * kk, g, preferred_element_type=jnp.float32)
        o_ref[...] = jnp.sqrt(jnp.max(kn2, axis=0, keepdims=True) / float(_C))

    return pl.pallas_call(
        kern,
        out_shape=jax.ShapeDtypeStruct((1, _HEADS), jnp.float32),
    )(k)


def _edge_math(qd, ks, kmaxs, be):
    etot = qd.shape[0]
    inv = 1.0 / float(np.sqrt(_C))

    def kern(qd_ref, ks_ref, km_ref, ex_ref):
        g = _head_onehot(_ED, _HEADS)
        q = qd_ref[...]
        k = ks_ref[...]
        score = jnp.dot(q * k, g, preferred_element_type=jnp.float32) * inv
        qn2 = jnp.dot(q * q, g, preferred_element_type=jnp.float32)
        mbound = jnp.sqrt(qn2) * km_ref[...]
        ex = jnp.exp(score - mbound)
        r16 = lax.broadcasted_iota(jnp.int32, (_HEADS, 128), 0)
        c16 = lax.broadcasted_iota(jnp.int32, (_HEADS, 128), 1)
        g16 = (c16 // 16 == r16).astype(jnp.float32)
        ex_ref[...] = jnp.dot(ex, g16, preferred_element_type=jnp.float32)

    return pl.pallas_call(
        kern,
        grid=(etot // be,),
        in_specs=[pl.BlockSpec((be, _ED), lambda i: (i, 0)),
                  pl.BlockSpec((be, _ED), lambda i: (i, 0)),
                  pl.BlockSpec((1, _HEADS), lambda i: (0, 0))],
        out_specs=pl.BlockSpec((be, 128), lambda i: (i, 0)),
        out_shape=jax.ShapeDtypeStruct((etot, 128), jnp.float32),
    )(qd, ks, kmaxs)


def _combine(out0, out1, den0, den1, s, bm):
    m = s.shape[0]

    def kern(o0_ref, o1_ref, d0_ref, d1_ref, s_ref, h_ref):
        r = lax.broadcasted_iota(jnp.int32, (128, 128), 0)
        c = lax.broadcasted_iota(jnp.int32, (128, 128), 1)
        realign = jnp.where(r // 16 == c // 16, 1.0 / 16.0, 0.0)
        d = d0_ref[...] + d1_ref[...]
        dexp = jnp.dot(d, realign, preferred_element_type=jnp.float32) + 1e-16
        attn = jnp.concatenate([o0_ref[...] / dexp, o1_ref[...] / dexp], axis=1)
        rp = lax.broadcasted_iota(jnp.int32, (_ED, _ED), 0)
        cp = lax.broadcasted_iota(jnp.int32, (_ED, _ED), 1)
        orig = 32 * ((rp % 128) // 16) + (rp % 16) + 16 * (rp // 128)
        pmat = (cp == orig).astype(jnp.float32)
        h_ref[...] = jnp.dot(attn, pmat,
                             preferred_element_type=jnp.float32) + s_ref[...]

    return pl.pallas_call(
        kern,
        grid=(m // bm,),
        in_specs=[pl.BlockSpec((bm, 128), lambda i: (i, 0)),
                  pl.BlockSpec((bm, 128), lambda i: (i, 0)),
                  pl.BlockSpec((bm, 128), lambda i: (i, 0)),
                  pl.BlockSpec((bm, 128), lambda i: (i, 0)),
                  pl.BlockSpec((bm, _ED), lambda i: (i, 0))],
        out_specs=pl.BlockSpec((bm, _ED), lambda i: (i, 0)),
        out_shape=jax.ShapeDtypeStruct((m, _ED), jnp.float32),
    )(out0, out1, den0, den1, s)


def _gather2(q, k, dst, src):
    etot = dst.shape[0]
    per_w = etot // _NW
    nch = per_w // _CH
    mesh = plsc.VectorSubcoreMesh(core_axis_name="c", subcore_axis_name="s")
    oshp = jax.ShapeDtypeStruct((etot, _ED), jnp.float32)

    @functools.partial(
        pl.kernel, mesh=mesh,
        out_type=[oshp, oshp],
        scratch_types=[pltpu.VMEM((_CH,), jnp.int32),
                       pltpu.VMEM((_CH,), jnp.int32),
                       pltpu.VMEM((_CH,), jnp.int32),
                       pltpu.VMEM((_CH,), jnp.int32),
                       pltpu.VMEM((_CH, _ED), jnp.float32),
                       pltpu.VMEM((_CH, _ED), jnp.float32),
                       pltpu.VMEM((_CH, _ED), jnp.float32),
                       pltpu.VMEM((_CH, _ED), jnp.float32),
                       pltpu.SemaphoreType.DMA,
                       pltpu.SemaphoreType.DMA])
    def kern(q_hbm, k_hbm, dst_hbm, src_hbm, qd_hbm, ks_hbm,
             di_a, si_a, di_b, si_b, qbuf_a, kbuf_a, qbuf_b, kbuf_b,
             sem_a, sem_b):
        wid = lax.axis_index("s") * 2 + lax.axis_index("c")
        base0 = wid * per_w

        def fire(base, di_v, si_v, qbuf, kbuf, sem):
            pltpu.sync_copy(dst_hbm.at[pl.ds(base, _CH)], di_v)
            pltpu.sync_copy(src_hbm.at[pl.ds(base, _CH)], si_v)
            c1 = pltpu.async_copy(q_hbm.at[di_v], qbuf, sem)
            c2 = pltpu.async_copy(k_hbm.at[si_v], kbuf, sem)
            return c1, c2

        def drain(base, cpys, qbuf, kbuf):
            c1, c2 = cpys
            c1.wait()
            c2.wait()
            pltpu.sync_copy(qbuf, qd_hbm.at[pl.ds(base, _CH)])
            pltpu.sync_copy(kbuf, ks_hbm.at[pl.ds(base, _CH)])

        def pair(t, carry):
            base_a = base0 + (2 * t) * _CH
            base_b = base_a + _CH
            ca = fire(base_a, di_a, si_a, qbuf_a, kbuf_a, sem_a)
            cb = fire(base_b, di_b, si_b, qbuf_b, kbuf_b, sem_b)
            drain(base_a, ca, qbuf_a, kbuf_a)
            drain(base_b, cb, qbuf_b, kbuf_b)
            return carry

        lax.fori_loop(0, nch // 2, pair, 0)
        if nch % 2:
            base_t = base0 + (nch - 1) * _CH
            ct = fire(base_t, di_a, si_a, qbuf_a, kbuf_a, sem_a)
            drain(base_t, ct, qbuf_a, kbuf_a)

    return kern(q, k, dst, src)


def _scatter(v0, v1, ex, dst, src, np_):
    etot = dst.shape[0]
    per_s = etot // 16
    nch = per_s // _CH
    per_s2 = etot // 32
    nch2 = per_s2 // _CH
    rows_per_sub = np_ // 16
    mesh = plsc.VectorSubcoreMesh(core_axis_name="c", subcore_axis_name="s")
    zrows = jnp.zeros((16, 128), jnp.float32)
    oshp = jax.ShapeDtypeStruct((np_, 128), jnp.float32)

    @functools.partial(
        pl.kernel, mesh=mesh,
        out_type=[oshp, oshp, oshp, oshp],
        scratch_types=[pltpu.VMEM((_CH,), jnp.int32),
                       pltpu.VMEM((_CH,), jnp.int32),
                       pltpu.VMEM((_CH,), jnp.int32),
                       pltpu.VMEM((_CH,), jnp.int32),
                       pltpu.VMEM((_CH, 128), jnp.float32),
                       pltpu.VMEM((_CH, 128), jnp.float32),
                       pltpu.VMEM((_CH, 128), jnp.float32),
                       pltpu.VMEM((_CH, 128), jnp.float32),
                       pltpu.VMEM((16, 128), jnp.float32),
                       pltpu.VMEM_SHARED((np_, 128), jnp.float32),
                       pltpu.SemaphoreType.DMA,
                       pltpu.SemaphoreType.DMA,
                       pltpu.SemaphoreType.DMA])
    def kern(v0_hbm, v1_hbm, ex_hbm, dst_hbm, src_hbm, z_hbm,
             out0_hbm, out1_hbm, den0_hbm, den1_hbm,
             di_a, si_a, di_b, si_b, vbuf_a, exbuf_a, vbuf_b, exbuf_b,
             zbuf, acc_sh, sem_a, sem_b, sem_add):
        cc = lax.axis_index("c")
        ss = lax.axis_index("s")
        row0 = ss * rows_per_sub

        pltpu.sync_copy(z_hbm, zbuf)

        def zero_acc():
            def zbody(t, carry):
                pltpu.sync_copy(zbuf, acc_sh.at[pl.ds(row0 + t * 16, 16)])
                return carry
            lax.fori_loop(0, rows_per_sub // 16, zbody, 0)

        def mul_rows(vbuf, exbuf):
            def mul_row(e, carry2):
                for g in range(8):
                    sl = pl.ds(16 * g, 16)
                    vbuf[e, sl] = vbuf[e, sl] * exbuf[e, sl]
                return carry2
            lax.fori_loop(0, _CH, mul_row, 0)

        def accum_v(v_hbm):
            def fire(base, di_v, si_v, vbuf, exbuf, sem):
                pltpu.sync_copy(dst_hbm.at[pl.ds(base, _CH)], di_v)
                pltpu.sync_copy(src_hbm.at[pl.ds(base, _CH)], si_v)
                c1 = pltpu.async_copy(v_hbm.at[si_v], vbuf, sem)
                c2 = pltpu.async_copy(ex_hbm.at[pl.ds(base, _CH)], exbuf, sem)
                return c1, c2

            def pair(t, carry):
                base_a = ss * per_s + (2 * t) * _CH
                ca = fire(base_a, di_a, si_a, vbuf_a, exbuf_a, sem_a)
                cb = fire(base_a + _CH, di_b, si_b, vbuf_b, exbuf_b, sem_b)
                ca[0].wait()
                ca[1].wait()
                mul_rows(vbuf_a, exbuf_a)
                add_a = pltpu.async_copy(vbuf_a, acc_sh.at[di_a], sem_add,
                                         add=True)
                cb[0].wait()
                cb[1].wait()
                mul_rows(vbuf_b, exbuf_b)
                add_a.wait()
                add_b = pltpu.async_copy(vbuf_b, acc_sh.at[di_b], sem_add,
                                         add=True)
                add_b.wait()
                return carry

            lax.fori_loop(0, nch // 2, pair, 0)
            if nch % 2:
                base_t = ss * per_s + (nch - 1) * _CH
                ct = fire(base_t, di_a, si_a, vbuf_a, exbuf_a, sem_a)
                ct[0].wait()
                ct[1].wait()
                mul_rows(vbuf_a, exbuf_a)
                pltpu.sync_copy(vbuf_a, acc_sh.at[di_a], add=True)

        def accum_ex(base0):
            def fire(base, di_v, exbuf, sem):
                pltpu.sync_copy(dst_hbm.at[pl.ds(base, _CH)], di_v)
                return pltpu.async_copy(ex_hbm.at[pl.ds(base, _CH)], exbuf, sem)

            def pair(t, carry):
                base_a = base0 + (2 * t) * _CH
                ca = fire(base_a, di_a, exbuf_a, sem_a)
                cb = fire(base_a + _CH, di_b, exbuf_b, sem_b)
                ca.wait()
                pltpu.sync_copy(exbuf_a, acc_sh.at[di_a], add=True)
                cb.wait()
                pltpu.sync_copy(exbuf_b, acc_sh.at[di_b], add=True)
                return carry

            lax.fori_loop(0, nch2 // 2, pair, 0)
            if nch2 % 2:
                base_t = base0 + (nch2 - 1) * _CH
                ct = fire(base_t, di_a, exbuf_a, sem_a)
                ct.wait()
                pltpu.sync_copy(exbuf_a, acc_sh.at[di_a], add=True)

        def dump(dst_hbm_out):
            pltpu.sync_copy(acc_sh.at[pl.ds(row0, rows_per_sub)],
                            dst_hbm_out.at[pl.ds(row0, rows_per_sub)])

        zero_acc()
        plsc.subcore_barrier()

        @pl.when(cc == 0)
        def _():
            accum_v(v0_hbm)

        @pl.when(cc == 1)
        def _():
            accum_v(v1_hbm)

        plsc.subcore_barrier()

        @pl.when(cc == 0)
        def _():
            dump(out0_hbm)

        @pl.when(cc == 1)
        def _():
            dump(out1_hbm)

        plsc.subcore_barrier()

        zero_acc()
        plsc.subcore_barrier()

        @pl.when(cc == 0)
        def _():
            accum_ex(ss * per_s2)

        @pl.when(cc == 1)
        def _():
            accum_ex(etot // 2 + ss * per_s2)

        plsc.subcore_barrier()

        @pl.when(cc == 0)
        def _():
            dump(den0_hbm)

        @pl.when(cc == 1)
        def _():
            dump(den1_hbm)

    return kern(v0, v1, ex, dst, src, zrows)


def kernel(x, edge_index, w_in1, b_in1, w_in2, b_in2, w_in3, b_in3,
           wq, bq, wk, bk, wv, bv, ws, bs,
           w_o1, b_o1, w_o2, b_o2, w_o3, b_o3):
    n = x.shape[0]
    npad = ((n + 511) // 512) * 512
    nlayers = wq.shape[0]
    bm = 512

    xp = jnp.pad(x, ((0, npad - n), (0, 0)))
    src = edge_index[0]
    dst = edge_index[1]

    h = _mlp3(xp, w_in1, b_in1, w_in2, b_in2, w_in3, b_in3, bm)

    pperm = np.empty((_ED,), np.int32)
    for p in range(_ED):
        pperm[p] = 32 * ((p % 128) // 16) + (p % 16) + 16 * (p // 128)

    for l in range(nlayers):
        w4 = jnp.concatenate([wq[l], wk[l], wv[l][:, pperm], ws[l]], axis=1)
        b4 = jnp.concatenate([bq[l], bk[l], bv[l][pperm], bs[l]])
        q, k, v0, v1, s = _proj(h, w4, b4, bm)
        kmaxs = _kmax(k)
        qd, ksg = _gather2(q, k, dst, src)
        ex = _edge_math(qd, ksg, kmaxs, 1000)
        out0, out1, den0, den1 = _scatter(v0, v1, ex, dst, src, npad)
        h = _combine(out0, out1, den0, den1, s, bm)

    o = _mlp3(h, w_o1, b_o1, w_o2, b_o2, w_o3, b_o3, bm, final_norm=True)
    return o[:n]


# speedup vs baseline: 1.3214x; 1.0079x over previous
import functools

import jax
import jax.numpy as jnp
import numpy as np
from jax import lax
from jax.experimental import pallas as pl
from jax.experimental.pallas import tpu as pltpu
from jax.experimental.pallas import tpu_sc as plsc

_CH = 80
_NW = 32
_HEADS = 8
_C = 32
_ED = 256


def _head_onehot(ncols, nheads, transpose=False):
    if transpose:
        r = lax.broadcasted_iota(jnp.int32, (nheads, ncols), 1)
        c = lax.broadcasted_iota(jnp.int32, (nheads, ncols), 0)
    else:
        r = lax.broadcasted_iota(jnp.int32, (ncols, nheads), 0)
        c = lax.broadcasted_iota(jnp.int32, (ncols, nheads), 1)
    return (r // _C == c).astype(jnp.float32)


def _mlp3(a, w1, b1, w2, b2, w3, b3, bm, final_norm=False):
    m, k = a.shape
    n = w3.shape[1]

    def kern(a_ref, w1_ref, b1_ref, w2_ref, b2_ref, w3_ref, b3_ref, o_ref):
        h1 = jnp.maximum(jnp.dot(a_ref[...], w1_ref[...],
                                 preferred_element_type=jnp.float32)
                         + b1_ref[...], 0.0)
        h2 = jnp.maximum(jnp.dot(h1, w2_ref[...],
                                 preferred_element_type=jnp.float32)
                         + b2_ref[...], 0.0)
        o = jnp.dot(h2, w3_ref[...],
                    preferred_element_type=jnp.float32) + b3_ref[...]
        if final_norm:
            norm = jnp.sqrt(jnp.sum(o * o, axis=1, keepdims=True))
            o_ref[...] = o / jnp.maximum(norm, 1e-12)
        else:
            o_ref[...] = jnp.maximum(o, 0.0)

    k2 = w2.shape[0]
    k3 = w3.shape[0]
    return pl.pallas_call(
        kern,
        grid=(m // bm,),
        in_specs=[pl.BlockSpec((bm, k), lambda i: (i, 0)),
                  pl.BlockSpec((k, k2), lambda i: (0, 0)),
                  pl.BlockSpec((1, k2), lambda i: (0, 0)),
                  pl.BlockSpec((k2, k3), lambda i: (0, 0)),
                  pl.BlockSpec((1, k3), lambda i: (0, 0)),
                  pl.BlockSpec((k3, n), lambda i: (0, 0)),
                  pl.BlockSpec((1, n), lambda i: (0, 0))],
        out_specs=pl.BlockSpec((bm, n), lambda i: (i, 0)),
        out_shape=jax.ShapeDtypeStruct((m, n), jnp.float32),
    )(a, w1, b1.reshape(1, -1), w2, b2.reshape(1, -1), w3, b3.reshape(1, -1))


def _proj(h, w4, b4, bm):
    m = h.shape[0]

    def kern(h_ref, w_ref, b_ref, q_ref, k_ref, v0_ref, v1_ref, s_ref):
        p = jnp.dot(h_ref[...], w_ref[...], preferred_element_type=jnp.float32)
        p = p + b_ref[...]
        q_ref[...] = p[:, 0:256]
        k_ref[...] = p[:, 256:512]
        v0_ref[...] = p[:, 512:640]
        v1_ref[...] = p[:, 640:768]
        s_ref[...] = p[:, 768:1024]

    shp = jax.ShapeDtypeStruct((m, _ED), jnp.float32)
    hshp = jax.ShapeDtypeStruct((m, 128), jnp.float32)
    return pl.pallas_call(
        kern,
        grid=(m // bm,),
        in_specs=[pl.BlockSpec((bm, _ED), lambda i: (i, 0)),
                  pl.BlockSpec((_ED, 4 * _ED), lambda i: (0, 0)),
                  pl.BlockSpec((1, 4 * _ED), lambda i: (0, 0))],
        out_specs=[pl.BlockSpec((bm, _ED), lambda i: (i, 0)),
                   pl.BlockSpec((bm, _ED), lambda i: (i, 0)),
                   pl.BlockSpec((bm, 128), lambda i: (i, 0)),
                   pl.BlockSpec((bm, 128), lambda i: (i, 0)),
                   pl.BlockSpec((bm, _ED), lambda i: (i, 0))],
        out_shape=[shp, shp, hshp, hshp, shp],
    )(h, w4, b4.reshape(1, 4 * _ED))


def _kmax(k):
    m = k.shape[0]

    def kern(k_ref, o_ref):
        kk = k_ref[...]
        g = _head_onehot(_ED, _HEADS)
        kn2 = jnp.dot(kk * kk, g, preferred_element_type=jnp.float32)
        o_ref[...] = jnp.sqrt(jnp.max(kn2, axis=0, keepdims=True) / float(_C))

    return pl.pallas_call(
        kern,
        out_shape=jax.ShapeDtypeStruct((1, _HEADS), jnp.float32),
    )(k)


def _edge_math(qd, ks, kmaxs, be):
    etot = qd.shape[0]
    inv = 1.0 / float(np.sqrt(_C))

    def kern(qd_ref, ks_ref, km_ref, ex_ref):
        g = _head_onehot(_ED, _HEADS)
        q = qd_ref[...]
        k = ks_ref[...]
        score = jnp.dot(q * k, g, preferred_element_type=jnp.float32) * inv
        qn2 = jnp.dot(q * q, g, preferred_element_type=jnp.float32)
        mbound = jnp.sqrt(qn2) * km_ref[...]
        ex = jnp.exp(score - mbound)
        r16 = lax.broadcasted_iota(jnp.int32, (_HEADS, 128), 0)
        c16 = lax.broadcasted_iota(jnp.int32, (_HEADS, 128), 1)
        g16 = (c16 // 16 == r16).astype(jnp.float32)
        ex_ref[...] = jnp.dot(ex, g16, preferred_element_type=jnp.float32)

    return pl.pallas_call(
        kern,
        grid=(etot // be,),
        in_specs=[pl.BlockSpec((be, _ED), lambda i: (i, 0)),
                  pl.BlockSpec((be, _ED), lambda i: (i, 0)),
                  pl.BlockSpec((1, _HEADS), lambda i: (0, 0))],
        out_specs=pl.BlockSpec((be, 128), lambda i: (i, 0)),
        out_shape=jax.ShapeDtypeStruct((etot, 128), jnp.float32),
    )(qd, ks, kmaxs)


def _combine(out0, out1, den0, den1, s, bm):
    m = s.shape[0]

    def kern(o0_ref, o1_ref, d0_ref, d1_ref, s_ref, h_ref):
        r = lax.broadcasted_iota(jnp.int32, (128, 128), 0)
        c = lax.broadcasted_iota(jnp.int32, (128, 128), 1)
        realign = jnp.where(r // 16 == c // 16, 1.0 / 16.0, 0.0)
        d = d0_ref[...] + d1_ref[...]
        dexp = jnp.dot(d, realign, preferred_element_type=jnp.float32) + 1e-16
        attn = jnp.concatenate([o0_ref[...] / dexp, o1_ref[...] / dexp], axis=1)
        rp = lax.broadcasted_iota(jnp.int32, (_ED, _ED), 0)
        cp = lax.broadcasted_iota(jnp.int32, (_ED, _ED), 1)
        orig = 32 * ((rp % 128) // 16) + (rp % 16) + 16 * (rp // 128)
        pmat = (cp == orig).astype(jnp.float32)
        h_ref[...] = jnp.dot(attn, pmat,
                             preferred_element_type=jnp.float32) + s_ref[...]

    return pl.pallas_call(
        kern,
        grid=(m // bm,),
        in_specs=[pl.BlockSpec((bm, 128), lambda i: (i, 0)),
                  pl.BlockSpec((bm, 128), lambda i: (i, 0)),
                  pl.BlockSpec((bm, 128), lambda i: (i, 0)),
                  pl.BlockSpec((bm, 128), lambda i: (i, 0)),
                  pl.BlockSpec((bm, _ED), lambda i: (i, 0))],
        out_specs=pl.BlockSpec((bm, _ED), lambda i: (i, 0)),
        out_shape=jax.ShapeDtypeStruct((m, _ED), jnp.float32),
    )(out0, out1, den0, den1, s)


def _gather2(q, k, dst, src):
    etot = dst.shape[0]
    per_w = etot // _NW
    nch = per_w // _CH
    mesh = plsc.VectorSubcoreMesh(core_axis_name="c", subcore_axis_name="s")
    oshp = jax.ShapeDtypeStruct((etot, _ED), jnp.float32)

    group = 5
    big = group * _CH
    nbig = per_w // big
    assert per_w % big == 0 and nch == nbig * group

    @functools.partial(
        pl.kernel, mesh=mesh,
        out_type=[oshp, oshp],
        scratch_types=[pltpu.VMEM((big,), jnp.int32),
                       pltpu.VMEM((big,), jnp.int32),
                       pltpu.VMEM((_CH, _ED), jnp.float32),
                       pltpu.VMEM((_CH, _ED), jnp.float32),
                       pltpu.VMEM((_CH, _ED), jnp.float32),
                       pltpu.VMEM((_CH, _ED), jnp.float32),
                       pltpu.SemaphoreType.DMA,
                       pltpu.SemaphoreType.DMA])
    def kern(q_hbm, k_hbm, dst_hbm, src_hbm, qd_hbm, ks_hbm,
             di_v, si_v, qbuf_a, kbuf_a, qbuf_b, kbuf_b,
             sem_a, sem_b):
        wid = lax.axis_index("s") * 2 + lax.axis_index("c")
        base0 = wid * per_w
        bufs = [(qbuf_a, kbuf_a, sem_a), (qbuf_b, kbuf_b, sem_b)]

        def body(t, carry):
            gbase = base0 + t * big
            pltpu.sync_copy(dst_hbm.at[pl.ds(gbase, big)], di_v)
            pltpu.sync_copy(src_hbm.at[pl.ds(gbase, big)], si_v)

            def fire(c):
                qbuf, kbuf, sem = bufs[c % 2]
                sl = pl.ds(c * _CH, _CH)
                c1 = pltpu.async_copy(q_hbm.at[di_v.at[sl]], qbuf, sem)
                c2 = pltpu.async_copy(k_hbm.at[si_v.at[sl]], kbuf, sem)
                return c1, c2

            def drain(c, cpys):
                qbuf, kbuf, _ = bufs[c % 2]
                cpys[0].wait()
                cpys[1].wait()
                base = gbase + c * _CH
                pltpu.sync_copy(qbuf, qd_hbm.at[pl.ds(base, _CH)])
                pltpu.sync_copy(kbuf, ks_hbm.at[pl.ds(base, _CH)])

            h = [None] * group
            h[0] = fire(0)
            for c in range(1, group):
                h[c] = fire(c)
                drain(c - 1, h[c - 1])
            drain(group - 1, h[group - 1])
            return carry

        lax.fori_loop(0, nbig, body, 0)

    return kern(q, k, dst, src)


def _scatter(v0, v1, ex, dst, src, np_):
    etot = dst.shape[0]
    per_s = etot // 16
    nch = per_s // _CH
    per_s2 = etot // 32
    nch2 = per_s2 // _CH
    rows_per_sub = np_ // 16
    mesh = plsc.VectorSubcoreMesh(core_axis_name="c", subcore_axis_name="s")
    zrows = jnp.zeros((16, 128), jnp.float32)
    oshp = jax.ShapeDtypeStruct((np_, 128), jnp.float32)

    @functools.partial(
        pl.kernel, mesh=mesh,
        out_type=[oshp, oshp, oshp, oshp],
        scratch_types=[pltpu.VMEM((_CH,), jnp.int32),
                       pltpu.VMEM((_CH,), jnp.int32),
                       pltpu.VMEM((_CH,), jnp.int32),
                       pltpu.VMEM((_CH,), jnp.int32),
                       pltpu.VMEM((_CH, 128), jnp.float32),
                       pltpu.VMEM((_CH, 128), jnp.float32),
                       pltpu.VMEM((_CH, 128), jnp.float32),
                       pltpu.VMEM((_CH, 128), jnp.float32),
                       pltpu.VMEM((16, 128), jnp.float32),
                       pltpu.VMEM_SHARED((np_, 128), jnp.float32),
                       pltpu.SemaphoreType.DMA,
                       pltpu.SemaphoreType.DMA,
                       pltpu.SemaphoreType.DMA])
    def kern(v0_hbm, v1_hbm, ex_hbm, dst_hbm, src_hbm, z_hbm,
             out0_hbm, out1_hbm, den0_hbm, den1_hbm,
             di_a, si_a, di_b, si_b, vbuf_a, exbuf_a, vbuf_b, exbuf_b,
             zbuf, acc_sh, sem_a, sem_b, sem_add):
        cc = lax.axis_index("c")
        ss = lax.axis_index("s")
        row0 = ss * rows_per_sub

        pltpu.sync_copy(z_hbm, zbuf)

        def zero_acc():
            def zbody(t, carry):
                pltpu.sync_copy(zbuf, acc_sh.at[pl.ds(row0 + t * 16, 16)])
                return carry
            lax.fori_loop(0, rows_per_sub // 16, zbody, 0)

        def mul_rows(vbuf, exbuf):
            def mul_row(e, carry2):
                for g in range(8):
                    sl = pl.ds(16 * g, 16)
                    vbuf[e, sl] = vbuf[e, sl] * exbuf[e, sl]
                return carry2
            lax.fori_loop(0, _CH, mul_row, 0)

        def accum_v(v_hbm):
            def fire(base, di_v, si_v, vbuf, exbuf, sem):
                pltpu.sync_copy(dst_hbm.at[pl.ds(base, _CH)], di_v)
                pltpu.sync_copy(src_hbm.at[pl.ds(base, _CH)], si_v)
                c1 = pltpu.async_copy(v_hbm.at[si_v], vbuf, sem)
                c2 = pltpu.async_copy(ex_hbm.at[pl.ds(base, _CH)], exbuf, sem)
                return c1, c2

            def pair(t, carry):
                base_a = ss * per_s + (2 * t) * _CH
                ca = fire(base_a, di_a, si_a, vbuf_a, exbuf_a, sem_a)
                cb = fire(base_a + _CH, di_b, si_b, vbuf_b, exbuf_b, sem_b)
                ca[0].wait()
                ca[1].wait()
                mul_rows(vbuf_a, exbuf_a)
                add_a = pltpu.async_copy(vbuf_a, acc_sh.at[di_a], sem_add,
                                         add=True)
                cb[0].wait()
                cb[1].wait()
                mul_rows(vbuf_b, exbuf_b)
                add_a.wait()
                add_b = pltpu.async_copy(vbuf_b, acc_sh.at[di_b], sem_add,
                                         add=True)
                add_b.wait()
                return carry

            lax.fori_loop(0, nch // 2, pair, 0)
            if nch % 2:
                base_t = ss * per_s + (nch - 1) * _CH
                ct = fire(base_t, di_a, si_a, vbuf_a, exbuf_a, sem_a)
                ct[0].wait()
                ct[1].wait()
                mul_rows(vbuf_a, exbuf_a)
                pltpu.sync_copy(vbuf_a, acc_sh.at[di_a], add=True)

        def accum_ex(base0):
            def fire(base, di_v, exbuf, sem):
                pltpu.sync_copy(dst_hbm.at[pl.ds(base, _CH)], di_v)
                return pltpu.async_copy(ex_hbm.at[pl.ds(base, _CH)], exbuf, sem)

            def pair(t, carry):
                base_a = base0 + (2 * t) * _CH
                ca = fire(base_a, di_a, exbuf_a, sem_a)
                cb = fire(base_a + _CH, di_b, exbuf_b, sem_b)
                ca.wait()
                pltpu.sync_copy(exbuf_a, acc_sh.at[di_a], add=True)
                cb.wait()
                pltpu.sync_copy(exbuf_b, acc_sh.at[di_b], add=True)
                return carry

            lax.fori_loop(0, nch2 // 2, pair, 0)
            if nch2 % 2:
                base_t = base0 + (nch2 - 1) * _CH
                ct = fire(base_t, di_a, exbuf_a, sem_a)
                ct.wait()
                pltpu.sync_copy(exbuf_a, acc_sh.at[di_a], add=True)

        def dump(dst_hbm_out):
            pltpu.sync_copy(acc_sh.at[pl.ds(row0, rows_per_sub)],
                            dst_hbm_out.at[pl.ds(row0, rows_per_sub)])

        zero_acc()
        plsc.subcore_barrier()

        @pl.when(cc == 0)
        def _():
            accum_v(v0_hbm)

        @pl.when(cc == 1)
        def _():
            accum_v(v1_hbm)

        plsc.subcore_barrier()

        @pl.when(cc == 0)
        def _():
            dump(out0_hbm)

        @pl.when(cc == 1)
        def _():
            dump(out1_hbm)

        plsc.subcore_barrier()

        zero_acc()
        plsc.subcore_barrier()

        @pl.when(cc == 0)
        def _():
            accum_ex(ss * per_s2)

        @pl.when(cc == 1)
        def _():
            accum_ex(etot // 2 + ss * per_s2)

        plsc.subcore_barrier()

        @pl.when(cc == 0)
        def _():
            dump(den0_hbm)

        @pl.when(cc == 1)
        def _():
            dump(den1_hbm)

    return kern(v0, v1, ex, dst, src, zrows)


def kernel(x, edge_index, w_in1, b_in1, w_in2, b_in2, w_in3, b_in3,
           wq, bq, wk, bk, wv, bv, ws, bs,
           w_o1, b_o1, w_o2, b_o2, w_o3, b_o3):
    n = x.shape[0]
    npad = ((n + 511) // 512) * 512
    nlayers = wq.shape[0]
    bm = 512

    xp = jnp.pad(x, ((0, npad - n), (0, 0)))
    src = edge_index[0]
    dst = edge_index[1]

    h = _mlp3(xp, w_in1, b_in1, w_in2, b_in2, w_in3, b_in3, bm)

    pperm = np.empty((_ED,), np.int32)
    for p in range(_ED):
        pperm[p] = 32 * ((p % 128) // 16) + (p % 16) + 16 * (p // 128)

    for l in range(nlayers):
        w4 = jnp.concatenate([wq[l], wk[l], wv[l][:, pperm], ws[l]], axis=1)
        b4 = jnp.concatenate([bq[l], bk[l], bv[l][pperm], bs[l]])
        q, k, v0, v1, s = _proj(h, w4, b4, bm)
        kmaxs = _kmax(k)
        qd, ksg = _gather2(q, k, dst, src)
        ex = _edge_math(qd, ksg, kmaxs, 1000)
        out0, out1, den0, den1 = _scatter(v0, v1, ex, dst, src, npad)
        h = _combine(out0, out1, den0, den1, s, bm)

    o = _mlp3(h, w_o1, b_o1, w_o2, b_o2, w_o3, b_o3, bm, final_norm=True)
    return o[:n]
